# baseline (speedup 1.0000x reference)
import functools

import jax
import jax.numpy as jnp
from jax import lax
from jax.experimental import pallas as pl
from jax.experimental.pallas import tpu as pltpu
from jax.experimental.pallas import tpu_sc as plsc

N = 10000
E = 320000
D = 128
EPS = 1e-5

NC = 2
NS = 16
NW = NC * NS
CHUNK = 128
CPW = 80
BC = 16
NBLK = CPW // BC
E_PAD = NW * CPW * CHUNK
ACC_ROWS = 10240
ZROWS_PER_SUB = ACC_ROWS // NS


def _ln_body(x_ref, w_ref, b_ref, h_ref):
    x = x_ref[...]
    mean = jnp.mean(x)
    var = jnp.mean((x - mean) ** 2)
    inv = 1.0 / (jnp.sqrt(var) + EPS)
    h = (x - mean) * inv * w_ref[...] + b_ref[...]
    h_ref[...] = jnp.maximum(h, 0.0)


def _layernorm_relu(x, w, b):
    return pl.pallas_call(
        _ln_body,
        out_shape=jax.ShapeDtypeStruct((N, D), jnp.float32),
    )(x, w.reshape(1, D), b.reshape(1, D))


def _sc_body(h_hbm, src_hbm, dst_hbm, acc_out,
             src_idx, dst_idx, zidx, rows_v, acc_sh,
             sem):
    c = lax.axis_index("c")
    s = lax.axis_index("s")
    wid = s * NC + c

    zeros16 = jnp.zeros((16,), jnp.float32)

    @pl.loop(0, CHUNK)
    def _fill_z(i):
        for j in range(D // 16):
            rows_v[i, pl.ds(16 * j, 16)] = zeros16

    zbase = s * ZROWS_PER_SUB
    iota16 = lax.iota(jnp.int32, 16)

    def _set_zidx(row0):
        for k in range(CHUNK // 16):
            zidx[pl.ds(16 * k, 16)] = iota16 + (row0 + 16 * k)

    for b in range(ZROWS_PER_SUB // CHUNK):
        _set_zidx(zbase + CHUNK * b)
        pltpu.async_copy(rows_v, acc_sh.at[zidx], sem).wait()

    plsc.subcore_barrier()

    ibase = wid * CPW

    for b in range(NBLK):
        pltpu.sync_copy(src_hbm.at[pl.ds(ibase + b * BC, BC)], src_idx)
        pltpu.sync_copy(dst_hbm.at[pl.ds(ibase + b * BC, BC)], dst_idx)

        @pl.loop(0, BC)
        def _edge(j):
            pltpu.async_copy(h_hbm.at[src_idx.at[j]], rows_v, sem).wait()
            pltpu.async_copy(rows_v, acc_sh.at[dst_idx.at[j]], sem, add=True).wait()

    plsc.subcore_barrier()

    obase = c * ACC_ROWS + zbase

    for b in range(ZROWS_PER_SUB // CHUNK):
        off = CHUNK * b
        _set_zidx(zbase + off)
        pltpu.async_copy(acc_sh.at[zidx], rows_v, sem).wait()
        pltpu.sync_copy(rows_v, acc_out.at[pl.ds(obase + off, CHUNK)])


@functools.cache
def _sc_segment_sum_kernel(table_rows):
    return pl.kernel(
        _sc_body,
        out_type=jax.ShapeDtypeStruct((NC * ACC_ROWS, D), jnp.float32),
        mesh=plsc.VectorSubcoreMesh(core_axis_name="c", subcore_axis_name="s"),
        scratch_types=[
            pltpu.VMEM((BC, CHUNK), jnp.int32),
            pltpu.VMEM((BC, CHUNK), jnp.int32),
            pltpu.VMEM((CHUNK,), jnp.int32),
            pltpu.VMEM((CHUNK, D), jnp.float32),
            pltpu.VMEM_SHARED((ACC_ROWS, D), jnp.float32),
            pltpu.SemaphoreType.DMA,
        ],
    )


_RB = 1000


def _comb_body(acc_ref, cnt_ref, h_ref, wlt_ref, bl_ref, wrt_ref, out_ref):
    a = acc_ref[0] + acc_ref[1]
    cnt = cnt_ref[0, :, 0:1] + cnt_ref[1, :, 0:1]
    mean_agg = a / jnp.maximum(cnt, 1.0)
    h = h_ref[...]
    out_ref[...] = (
        jnp.dot(mean_agg, wlt_ref[...], preferred_element_type=jnp.float32)
        + bl_ref[...]
        + jnp.dot(h, wrt_ref[...], preferred_element_type=jnp.float32)
    )


def _combine(acc, cnt, h, W_l, b_l, W_r):
    grid = N // _RB
    return pl.pallas_call(
        _comb_body,
        grid=(grid,),
        in_specs=[
            pl.BlockSpec((NC, _RB, D), lambda i: (0, i, 0)),
            pl.BlockSpec((NC, _RB, D), lambda i: (0, i, 0)),
            pl.BlockSpec((_RB, D), lambda i: (i, 0)),
            pl.BlockSpec((D, D), lambda i: (0, 0)),
            pl.BlockSpec((1, D), lambda i: (0, 0)),
            pl.BlockSpec((D, D), lambda i: (0, 0)),
        ],
        out_specs=pl.BlockSpec((_RB, D), lambda i: (i, 0)),
        out_shape=jax.ShapeDtypeStruct((N, D), jnp.float32),
    )(acc, cnt, h, W_l.T, b_l.reshape(1, D), W_r.T)


def kernel(x, edge_index, ln_weight, ln_bias, W_l, b_l, W_r):
    h = _layernorm_relu(x, ln_weight, ln_bias)

    src = edge_index[0]
    dst = edge_index[1]
    pad = E_PAD - E
    src_p = jnp.concatenate([src, jnp.zeros((pad,), jnp.int32)])
    dst_p = jnp.concatenate([dst, jnp.full((pad,), N, jnp.int32)])
    src_p = src_p.reshape(NW * CPW, CHUNK)
    dst_p = dst_p.reshape(NW * CPW, CHUNK)

    acc_f = _sc_segment_sum_kernel(N)(h, src_p, dst_p)
    ones_tab = jnp.ones((8, D), jnp.float32)
    cnt_f = _sc_segment_sum_kernel(8)(ones_tab, jnp.zeros_like(src_p), dst_p)
    acc = acc_f.reshape(NC, ACC_ROWS, D)
    cnt = cnt_f.reshape(NC, ACC_ROWS, D)
    return _combine(acc, cnt, h, W_l, b_l, W_r)


# speedup vs baseline: 22.8590x; 22.8590x over previous
import functools

import jax
import jax.numpy as jnp
from jax import lax
from jax.experimental import pallas as pl
from jax.experimental.pallas import tpu as pltpu
from jax.experimental.pallas import tpu_sc as plsc

N = 10000
E = 320000
D = 128
EPS = 1e-5

NC = 2
NS = 16
NW = NC * NS
CHUNK = 128
CPW = 80
BC = 16
NBLK = CPW // BC
E_PAD = NW * CPW * CHUNK
ACC_ROWS = 10240
ZROWS_PER_SUB = ACC_ROWS // NS


def _ln_body(x_ref, w_ref, b_ref, h_ref):
    x = x_ref[...]
    mean = jnp.mean(x)
    var = jnp.mean((x - mean) ** 2)
    inv = 1.0 / (jnp.sqrt(var) + EPS)
    h = (x - mean) * inv * w_ref[...] + b_ref[...]
    h_ref[...] = jnp.maximum(h, 0.0)


def _layernorm_relu(x, w, b):
    return pl.pallas_call(
        _ln_body,
        out_shape=jax.ShapeDtypeStruct((N, D), jnp.float32),
    )(x, w.reshape(1, D), b.reshape(1, D))


def _sc_body(do_gather, h_hbm, src_hbm, dst_hbm, acc_out,
             src_idx, dst_idx, zidx, rows_v, rows_w, acc_sh,
             sem, sem2, sem3):
    c = lax.axis_index("c")
    s = lax.axis_index("s")
    wid = s * NC + c

    zeros16 = jnp.zeros((16,), jnp.float32)

    @pl.loop(0, CHUNK)
    def _fill_z(i):
        for j in range(D // 16):
            rows_v[i, pl.ds(16 * j, 16)] = zeros16

    zbase = s * ZROWS_PER_SUB
    iota16 = lax.iota(jnp.int32, 16)

    def _set_zidx(row0):
        for k in range(CHUNK // 16):
            zidx[pl.ds(16 * k, 16)] = iota16 + (row0 + 16 * k)

    for b in range(ZROWS_PER_SUB // CHUNK):
        _set_zidx(zbase + CHUNK * b)
        pltpu.async_copy(rows_v, acc_sh.at[zidx], sem).wait()

    if not do_gather:
        ones16f = jnp.ones((16,), jnp.float32)

        @pl.loop(0, CHUNK)
        def _fill_one(i):
            for j in range(D // 16):
                rows_v[i, pl.ds(16 * j, 16)] = ones16f

    plsc.subcore_barrier()

    ibase = wid * CPW

    for b in range(NBLK):
        pltpu.sync_copy(dst_hbm.at[pl.ds(ibase + b * BC, BC)], dst_idx)
        if do_gather:
            pltpu.sync_copy(src_hbm.at[pl.ds(ibase + b * BC, BC)], src_idx)

            @pl.loop(0, BC // 2)
            def _edge(j):
                ga = pltpu.async_copy(h_hbm.at[src_idx.at[2 * j]], rows_v, sem)
                gb = pltpu.async_copy(h_hbm.at[src_idx.at[2 * j + 1]], rows_w, sem2)
                ga.wait()
                sa = pltpu.async_copy(rows_v, acc_sh.at[dst_idx.at[2 * j]], sem3, add=True)
                gb.wait()
                sa.wait()
                pltpu.async_copy(rows_w, acc_sh.at[dst_idx.at[2 * j + 1]], sem3, add=True).wait()
        else:
            @pl.loop(0, BC // 4)
            def _cnt(j):
                cps = [
                    pltpu.async_copy(rows_v, acc_sh.at[dst_idx.at[4 * j + k]],
                                     sem3, add=True)
                    for k in range(4)
                ]
                for cp in cps:
                    cp.wait()

    plsc.subcore_barrier()

    obase = c * ACC_ROWS + zbase

    for b in range(ZROWS_PER_SUB // CHUNK):
        off = CHUNK * b
        _set_zidx(zbase + off)
        pltpu.async_copy(acc_sh.at[zidx], rows_v, sem).wait()
        pltpu.sync_copy(rows_v, acc_out.at[pl.ds(obase + off, CHUNK)])


@functools.cache
def _sc_segment_sum_kernel(do_gather):
    return pl.kernel(
        functools.partial(_sc_body, do_gather),
        out_type=jax.ShapeDtypeStruct((NC * ACC_ROWS, D), jnp.float32),
        mesh=plsc.VectorSubcoreMesh(core_axis_name="c", subcore_axis_name="s"),
        scratch_types=[
            pltpu.VMEM((BC, CHUNK), jnp.int32),
            pltpu.VMEM((BC, CHUNK), jnp.int32),
            pltpu.VMEM((CHUNK,), jnp.int32),
            pltpu.VMEM((CHUNK, D), jnp.float32),
            pltpu.VMEM((CHUNK, D), jnp.float32),
            pltpu.VMEM_SHARED((ACC_ROWS, D), jnp.float32),
            pltpu.SemaphoreType.DMA,
            pltpu.SemaphoreType.DMA,
            pltpu.SemaphoreType.DMA,
        ],
    )


_RB = 1000


def _comb_body(acc_ref, cnt_ref, h_ref, wlt_ref, bl_ref, wrt_ref, out_ref):
    a = acc_ref[0] + acc_ref[1]
    cnt = cnt_ref[0, :, 0:1] + cnt_ref[1, :, 0:1]
    mean_agg = a / jnp.maximum(cnt, 1.0)
    h = h_ref[...]
    out_ref[...] = (
        jnp.dot(mean_agg, wlt_ref[...], preferred_element_type=jnp.float32)
        + bl_ref[...]
        + jnp.dot(h, wrt_ref[...], preferred_element_type=jnp.float32)
    )


def _combine(acc, cnt, h, W_l, b_l, W_r):
    grid = N // _RB
    return pl.pallas_call(
        _comb_body,
        grid=(grid,),
        in_specs=[
            pl.BlockSpec((NC, _RB, D), lambda i: (0, i, 0)),
            pl.BlockSpec((NC, _RB, D), lambda i: (0, i, 0)),
            pl.BlockSpec((_RB, D), lambda i: (i, 0)),
            pl.BlockSpec((D, D), lambda i: (0, 0)),
            pl.BlockSpec((1, D), lambda i: (0, 0)),
            pl.BlockSpec((D, D), lambda i: (0, 0)),
        ],
        out_specs=pl.BlockSpec((_RB, D), lambda i: (i, 0)),
        out_shape=jax.ShapeDtypeStruct((N, D), jnp.float32),
    )(acc, cnt, h, W_l.T, b_l.reshape(1, D), W_r.T)


def kernel(x, edge_index, ln_weight, ln_bias, W_l, b_l, W_r):
    h = _layernorm_relu(x, ln_weight, ln_bias)

    src = edge_index[0]
    dst = edge_index[1]
    pad = E_PAD - E
    src_p = jnp.concatenate([src, jnp.zeros((pad,), jnp.int32)])
    dst_p = jnp.concatenate([dst, jnp.full((pad,), N, jnp.int32)])
    src_p = src_p.reshape(NW * CPW, CHUNK)
    dst_p = dst_p.reshape(NW * CPW, CHUNK)

    acc_f = _sc_segment_sum_kernel(True)(h, src_p, dst_p)
    ones_tab = jnp.ones((8, D), jnp.float32)
    cnt_f = _sc_segment_sum_kernel(False)(ones_tab, src_p, dst_p)
    acc = acc_f.reshape(NC, ACC_ROWS, D)
    cnt = cnt_f.reshape(NC, ACC_ROWS, D)
    return _combine(acc, cnt, h, W_l, b_l, W_r)
